# Initial kernel scaffold; baseline (speedup 1.0000x reference)
import jax
import jax.numpy as jnp
from jax.experimental import pallas as pl


def kernel(x, edge_index, W_self1, W_neigh1, b1, W_self2, W_neigh2, b2):
    raise NotImplementedError("write your pallas kernel here")


# speedup vs baseline: 5.1059x; 5.1059x over previous
import functools

import jax
import jax.numpy as jnp
from jax import lax
from jax.experimental import pallas as pl
from jax.experimental.pallas import tpu as pltpu
from jax.experimental.pallas import tpu_sc as plsc

_N = 10000
_E = 320000
_D = 128

_NC = 2
_NS = 16
_NW = _NC * _NS
_CHUNK = 128
_EPT = _E // _NW
_CHUNKS = (_EPT + _CHUNK - 1) // _CHUNK
_EPT_PAD = _CHUNKS * _CHUNK
_ACC_ROWS = 10112
_ZROWS = _ACC_ROWS // _NS

_f32 = jnp.float32


def _sc_body(hw, src, dst, zeros, agg_out, src_v, dst_v, rows_v,
             acc_sh, gsem):
    cid = lax.axis_index("c")
    sid = lax.axis_index("s")
    wid = sid * _NC + cid

    pltpu.sync_copy(zeros, acc_sh.at[pl.ds(sid * _ZROWS, _ZROWS)])
    pltpu.sync_copy(src.at[wid], src_v)
    pltpu.sync_copy(dst.at[wid], dst_v)
    plsc.subcore_barrier()

    def step(j, carry):
        pltpu.async_copy(hw.at[src_v.at[j]], rows_v, gsem).wait()
        pltpu.sync_copy(rows_v, acc_sh.at[dst_v.at[j]], add=True)
        return carry

    lax.fori_loop(0, _CHUNKS, step, 0)
    plsc.subcore_barrier()

    pltpu.sync_copy(acc_sh.at[pl.ds(sid * _ZROWS, _ZROWS)],
                    agg_out.at[cid, pl.ds(sid * _ZROWS, _ZROWS)])


def _make_sc():
    mesh = plsc.VectorSubcoreMesh(core_axis_name="c", subcore_axis_name="s")
    out_type = jax.ShapeDtypeStruct((_NC, _ACC_ROWS, _D), _f32)
    scratch = [
        pltpu.VMEM((_CHUNKS, _CHUNK), jnp.int32),
        pltpu.VMEM((_CHUNKS, _CHUNK), jnp.int32),
        pltpu.VMEM((_CHUNK, _D), _f32),
        pltpu.VMEM_SHARED((_ACC_ROWS, _D), _f32),
        pltpu.SemaphoreType.DMA,
    ]
    return pl.kernel(_sc_body, out_type=out_type, mesh=mesh,
                     scratch_types=scratch)


def _deg_body(dst, zeros, ones, deg_out, dst_v, ones_v, deg_sh):
    cid = lax.axis_index("c")
    sid = lax.axis_index("s")
    wid = sid * _NC + cid

    pltpu.sync_copy(zeros, deg_sh.at[pl.ds(sid * _ZROWS, _ZROWS)])
    pltpu.sync_copy(dst.at[wid], dst_v)
    pltpu.sync_copy(ones, ones_v)
    plsc.subcore_barrier()

    def step(j, carry):
        pltpu.sync_copy(ones_v, deg_sh.at[dst_v.at[j]], add=True)
        return carry

    lax.fori_loop(0, _CHUNKS, step, 0)
    plsc.subcore_barrier()

    pltpu.sync_copy(deg_sh.at[pl.ds(sid * _ZROWS, _ZROWS)],
                    deg_out.at[cid, pl.ds(sid * _ZROWS, _ZROWS)])


def _make_deg():
    mesh = plsc.VectorSubcoreMesh(core_axis_name="c", subcore_axis_name="s")
    out_type = jax.ShapeDtypeStruct((_NC, _ACC_ROWS, _D), _f32)
    scratch = [
        pltpu.VMEM((_CHUNKS, _CHUNK), jnp.int32),
        pltpu.VMEM((_CHUNK, _D), _f32),
        pltpu.VMEM_SHARED((_ACC_ROWS, _D), _f32),
    ]
    return pl.kernel(_deg_body, out_type=out_type, mesh=mesh,
                     scratch_types=scratch)


_BR = 2000


def _mm2_body(x_ref, wa_ref, wb_ref, a_ref, b_ref):
    x = x_ref[...]
    dn = (((1,), (1,)), ((), ()))
    a_ref[...] = lax.dot_general(x, wa_ref[...], dn,
                                 preferred_element_type=_f32)
    b_ref[...] = lax.dot_general(x, wb_ref[...], dn,
                                 preferred_element_type=_f32)


def _mm2(x, wa, wb):
    grid = (_N // _BR,)
    blk_x = pl.BlockSpec((_BR, _D), lambda i: (i, 0))
    blk_w = pl.BlockSpec((_D, _D), lambda i: (0, 0))
    return pl.pallas_call(
        _mm2_body,
        grid=grid,
        in_specs=[blk_x, blk_w, blk_w],
        out_specs=[blk_x, blk_x],
        out_shape=[jax.ShapeDtypeStruct((_N, _D), _f32)] * 2,
    )(x, wa, wb)


def _mid_body(xs_ref, agg_ref, deg_ref, b_ref, wa_ref, wb_ref,
              a_ref, b_out_ref):
    deg = deg_ref[0, :, 0:1] + deg_ref[1, :, 0:1]
    recip = 1.0 / jnp.maximum(deg, 1.0)
    h = xs_ref[...] + (agg_ref[0] + agg_ref[1]) * recip + b_ref[...]
    h = jnp.maximum(h, 0.0)
    dn = (((1,), (1,)), ((), ()))
    a_ref[...] = lax.dot_general(h, wa_ref[...], dn,
                                 preferred_element_type=_f32)
    b_out_ref[...] = lax.dot_general(h, wb_ref[...], dn,
                                     preferred_element_type=_f32)


def _mid(xs, agg, deg, b, wa, wb):
    grid = (_N // _BR,)
    blk_r = pl.BlockSpec((_BR, _D), lambda i: (i, 0))
    blk_a = pl.BlockSpec((_NC, _BR, _D), lambda i: (0, i, 0))
    blk_d = pl.BlockSpec((_NC, _BR, 16), lambda i: (0, i, 0))
    blk_b = pl.BlockSpec((1, _D), lambda i: (0, 0))
    blk_w = pl.BlockSpec((_D, _D), lambda i: (0, 0))
    return pl.pallas_call(
        _mid_body,
        grid=grid,
        in_specs=[blk_r, blk_a, blk_d, blk_b, blk_w, blk_w],
        out_specs=[blk_r, blk_r],
        out_shape=[jax.ShapeDtypeStruct((_N, _D), _f32)] * 2,
    )(xs, agg, deg, b, wa, wb)


def _fin_body(xs_ref, agg_ref, deg_ref, b_ref, o_ref):
    deg = deg_ref[0, :, 0:1] + deg_ref[1, :, 0:1]
    recip = 1.0 / jnp.maximum(deg, 1.0)
    o_ref[...] = xs_ref[...] + (agg_ref[0] + agg_ref[1]) * recip + b_ref[...]


def _fin(xs, agg, deg, b):
    grid = (_N // _BR,)
    blk_r = pl.BlockSpec((_BR, _D), lambda i: (i, 0))
    blk_a = pl.BlockSpec((_NC, _BR, _D), lambda i: (0, i, 0))
    blk_d = pl.BlockSpec((_NC, _BR, 16), lambda i: (0, i, 0))
    blk_b = pl.BlockSpec((1, _D), lambda i: (0, 0))
    return pl.pallas_call(
        _fin_body,
        grid=grid,
        in_specs=[blk_r, blk_a, blk_d, blk_b],
        out_specs=blk_r,
        out_shape=jax.ShapeDtypeStruct((_N, _D), _f32),
    )(xs, agg, deg, b)


def kernel(x, edge_index, W_self1, W_neigh1, b1, W_self2, W_neigh2, b2):
    src = edge_index[0].reshape(_NW, _EPT)
    dst = edge_index[1].reshape(_NW, _EPT)
    pad = _EPT_PAD - _EPT
    src_p = jnp.concatenate(
        [src, jnp.zeros((_NW, pad), jnp.int32)], axis=1
    ).reshape(_NW, _CHUNKS, _CHUNK)
    dst_p = jnp.concatenate(
        [dst, jnp.full((_NW, pad), _N, jnp.int32)], axis=1
    ).reshape(_NW, _CHUNKS, _CHUNK)

    zeros = jnp.zeros((_ZROWS, _D), _f32)
    ones = jnp.ones((_CHUNK, _D), _f32)

    xs1, hw1 = _mm2(x, W_self1, W_neigh1)
    deg = _make_deg()(dst_p, zeros, ones)[:, :, :16]
    agg1 = _make_sc()(hw1, src_p, dst_p, zeros)
    xs2, hw2 = _mid(xs1, agg1, deg, b1.reshape(1, _D), W_self2, W_neigh2)
    agg2 = _make_sc()(hw2, src_p, dst_p, zeros)
    return _fin(xs2, agg2, deg, b2.reshape(1, _D))
